# baseline (speedup 1.0000x reference)
import functools

import jax
import jax.numpy as jnp
from jax import lax
from jax.experimental import pallas as pl
from jax.experimental.pallas import tpu as pltpu
from jax.experimental.pallas import tpu_sc as plsc

_C = 128
_NC, _NS = 2, 16
_NW = _NC * _NS
_CHUNK = 256


def _sc_body(cube_hbm, const_hbm, out_hbm, buf, constv):
    wid = lax.axis_index("s") * _NC + lax.axis_index("c")
    rows_per_w = cube_hbm.shape[0] // _NW
    base = wid * rows_per_w
    pltpu.sync_copy(const_hbm, constv)
    nchunk = rows_per_w // _CHUNK

    def chunk_body(i, carry):
        start = base + i * _CHUNK
        pltpu.sync_copy(cube_hbm.at[pl.ds(start, _CHUNK), :], buf)

        def row_body(r, c2):
            xs = []
            x = buf[r, pl.ds(0, 16)]
            xs.append(x)
            le = x <= 1.0
            eq = x == 1.0
            for j in range(1, 8):
                x = buf[r, pl.ds(j * 16, 16)]
                xs.append(x)
                le = jnp.logical_and(le, x <= 1.0)
                eq = jnp.logical_or(eq, x == 1.0)
            n_le = plsc.all_reduce_population_count(le)
            n_eq = plsc.all_reduce_population_count(eq)
            keep16 = jnp.logical_and(n_le == 16, n_eq > 0)
            for j in range(8):
                buf[r, pl.ds(j * 16, 16)] = jnp.where(
                    keep16, xs[j], constv[pl.ds(j * 16, 16)])
            return c2

        lax.fori_loop(0, _CHUNK, row_body, 0)
        pltpu.sync_copy(buf, out_hbm.at[pl.ds(start, _CHUNK), :])
        return carry

    lax.fori_loop(0, nchunk, chunk_body, 0)


def kernel(cayley_cube, constant_distribution):
    b, n, _, c = cayley_cube.shape
    rows = b * n * n
    flat = cayley_cube.reshape(rows, c)
    mesh = plsc.VectorSubcoreMesh(
        core_axis_name="c", subcore_axis_name="s",
        num_cores=_NC, num_subcores=_NS,
    )
    out = pl.kernel(
        _sc_body,
        out_type=jax.ShapeDtypeStruct((rows, c), jnp.float32),
        mesh=mesh,
        compiler_params=pltpu.CompilerParams(needs_layout_passes=False),
        scratch_types=[
            pltpu.VMEM((_CHUNK, c), jnp.float32),
            pltpu.VMEM((c,), jnp.float32),
        ],
    )(flat, constant_distribution)
    return out.reshape(b, n, n, c)


# speedup vs baseline: 3.3812x; 3.3812x over previous
import jax
import jax.numpy as jnp
from jax import lax
from jax.experimental import pallas as pl
from jax.experimental.pallas import tpu as pltpu
from jax.experimental.pallas import tpu_sc as plsc

_C = 128
_NC, _NS = 2, 16
_NW = _NC * _NS
_CHUNK = 128
_NBUF = 4
_PF = 2


def _row_pass(buf, constv, r):
    xs = [buf[r, pl.ds(j * 16, 16)] for j in range(8)]
    m = xs[0]
    for j in range(1, 8):
        m = jnp.maximum(m, xs[j])
    n_le = plsc.all_reduce_population_count(m <= 1.0)
    n_eq = plsc.all_reduce_population_count(m == 1.0)
    keep16 = jnp.logical_and(n_le == 16, n_eq > 0)
    for j in range(8):
        buf[r, pl.ds(j * 16, 16)] = jnp.where(
            keep16, xs[j], constv[pl.ds(j * 16, 16)])


def _sc_body(cube_hbm, const_hbm, out_hbm, bufs, constv, in_sems, out_sems):
    wid = lax.axis_index("s") * _NC + lax.axis_index("c")
    rows_per_w = cube_hbm.shape[0] // _NW
    base = wid * rows_per_w
    nchunk = rows_per_w // _CHUNK
    pltpu.sync_copy(const_hbm, constv)

    def in_slice(idx):
        return cube_hbm.at[pl.ds(base + idx * _CHUNK, _CHUNK), :]

    def out_slice(idx):
        return out_hbm.at[pl.ds(base + idx * _CHUNK, _CHUNK), :]

    for b in range(_PF):
        pltpu.async_copy(in_slice(b), bufs[b], in_sems[b])

    def super_body(i):
        for b in range(_NBUF):
            idx = i + b
            pf = idx + _PF
            bpf = (b + _PF) % _NBUF

            @pl.when(pf >= _NBUF)
            def _():
                pltpu.make_async_copy(
                    bufs[bpf], out_slice(0), out_sems[bpf]).wait()

            @pl.when(pf < nchunk)
            def _():
                pltpu.async_copy(in_slice(pf), bufs[bpf], in_sems[bpf])

            pltpu.make_async_copy(in_slice(0), bufs[b], in_sems[b]).wait()
            plsc.parallel_loop(0, _CHUNK, 1, unroll=4)(
                lambda r: _row_pass(bufs[b], constv, r))
            pltpu.async_copy(bufs[b], out_slice(idx), out_sems[b])

    pl.loop(0, nchunk, step=_NBUF)(super_body)
    for b in range(_NBUF - _PF, _NBUF):
        pltpu.make_async_copy(bufs[b], out_slice(0), out_sems[b]).wait()


def kernel(cayley_cube, constant_distribution):
    b, n, _, c = cayley_cube.shape
    rows = b * n * n
    flat = cayley_cube.reshape(rows, c)
    mesh = plsc.VectorSubcoreMesh(
        core_axis_name="c", subcore_axis_name="s",
        num_cores=_NC, num_subcores=_NS,
    )
    out = pl.kernel(
        _sc_body,
        out_type=jax.ShapeDtypeStruct((rows, c), jnp.float32),
        mesh=mesh,
        compiler_params=pltpu.CompilerParams(needs_layout_passes=False),
        scratch_types=[
            [pltpu.VMEM((_CHUNK, c), jnp.float32) for _ in range(_NBUF)],
            pltpu.VMEM((c,), jnp.float32),
            [pltpu.SemaphoreType.DMA for _ in range(_NBUF)],
            [pltpu.SemaphoreType.DMA for _ in range(_NBUF)],
        ],
    )(flat, constant_distribution)
    return out.reshape(b, n, n, c)
